# Initial kernel scaffold; baseline (speedup 1.0000x reference)
import jax
import jax.numpy as jnp
from jax.experimental import pallas as pl


def kernel(x, edge_index, W):
    raise NotImplementedError("write your pallas kernel here")


# speedup vs baseline: 90.2192x; 90.2192x over previous
import jax
import jax.numpy as jnp
from jax import lax
from jax.experimental import pallas as pl
from jax.experimental.pallas import tpu as pltpu
from jax.experimental.pallas import tpu_sc as plsc

N = 99999
E = 6399936
EPAD = 6400000
PER_W = 200000
CHUNK = 4000
NCHUNK = PER_W // CHUNK
NVEC = CHUNK // 16
NPACK = 50000
NOUT = 33333
ACC = 33536
COLS = ACC // 16
NCV = COLS // 16


def _sc_body(xp_hbm, src_hbm, dst_hbm, w_hbm, part_hbm,
             xp_v, src_v, dst_v, acc_v, w_v, out_v, shr):
    cid = lax.axis_index("c")
    sid = lax.axis_index("s")
    wid = sid * 2 + cid

    pltpu.sync_copy(xp_hbm, xp_v)
    pltpu.sync_copy(w_hbm, w_v)

    zero = jnp.zeros((16,), jnp.float32)

    def zbody(j, carry):
        acc_v[pl.ds(j * 16, 16)] = zero
        return carry
    lax.fori_loop(0, COLS, zbody, 0)

    ebase = wid * PER_W

    def chunk_body(g, carry):
        base = ebase + g * CHUNK
        pltpu.sync_copy(src_hbm.at[pl.ds(base, CHUNK)], src_v)
        pltpu.sync_copy(dst_hbm.at[pl.ds(base, CHUNK)], dst_v)

        def inner(i, c2):
            o = i * 16
            s16 = src_v[pl.ds(o, 16)]
            d16 = dst_v[pl.ds(o, 16)]
            pk = plsc.load_gather(xp_v, [s16 >> 1])
            bits = jnp.where((s16 & 1) == 1, pk & jnp.int32(-65536), pk << 16)
            val = plsc.bitcast(bits, jnp.float32)
            q = lax.div(d16, jnp.int32(3))
            msk = (d16 - q * 3) == 0
            plsc.addupdate_scatter(acc_v, [q], val, mask=msk)
            return c2
        lax.fori_loop(0, NVEC, inner, 0)
        return carry
    lax.fori_loop(0, NCHUNK, chunk_body, 0)

    pltpu.sync_copy(acc_v, shr.at[pl.ds(sid * ACC, ACC)])
    plsc.subcore_barrier()

    colbase = sid * COLS
    for p in range(16):
        pltpu.sync_copy(shr.at[pl.ds(p * ACC + colbase, COLS)],
                        acc_v.at[pl.ds(p * COLS, COLS)])

    wv = w_v[...]

    def rbody(j, carry):
        o = j * 16
        t = acc_v[pl.ds(o, 16)]
        for p in range(1, 16):
            t = t + acc_v[pl.ds(p * COLS + o, 16)]
        out_v[pl.ds(o, 16)] = t * wv
        return carry
    lax.fori_loop(0, NCV, rbody, 0)

    pltpu.sync_copy(out_v, part_hbm.at[pl.ds(cid * ACC + colbase, COLS)])


def _combine_body(p_ref, o_ref):
    o_ref[...] = p_ref[:ACC] + p_ref[ACC:]


def kernel(x, edge_index, W):
    xb = x.reshape(-1).astype(jnp.bfloat16)
    xb = jnp.concatenate([xb, jnp.zeros((1,), jnp.bfloat16)])
    xp = lax.bitcast_convert_type(xb.reshape(NPACK, 2), jnp.int32)

    pad = EPAD - E
    src = jnp.concatenate([edge_index[0], jnp.zeros((pad,), jnp.int32)])
    dst = jnp.concatenate([edge_index[1], jnp.ones((pad,), jnp.int32)])
    wvec = jnp.broadcast_to(W.reshape(()), (16,)).astype(jnp.float32)

    mesh = plsc.VectorSubcoreMesh(core_axis_name="c", subcore_axis_name="s",
                                  num_cores=2, num_subcores=16)
    part = pl.kernel(
        _sc_body,
        out_type=jax.ShapeDtypeStruct((2 * ACC,), jnp.float32),
        mesh=mesh,
        compiler_params=pltpu.CompilerParams(needs_layout_passes=False),
        scratch_types=[
            pltpu.VMEM((NPACK,), jnp.int32),
            pltpu.VMEM((CHUNK,), jnp.int32),
            pltpu.VMEM((CHUNK,), jnp.int32),
            pltpu.VMEM((ACC,), jnp.float32),
            pltpu.VMEM((16,), jnp.float32),
            pltpu.VMEM((COLS,), jnp.float32),
            pltpu.VMEM_SHARED((16 * ACC,), jnp.float32),
        ],
    )(xp, src, dst, wvec)

    out = pl.pallas_call(
        _combine_body,
        out_shape=jax.ShapeDtypeStruct((ACC,), jnp.float32),
    )(part)
    return out[:NOUT]


# speedup vs baseline: 90.2675x; 1.0005x over previous
import jax
import jax.numpy as jnp
from jax import lax
from jax.experimental import pallas as pl
from jax.experimental.pallas import tpu as pltpu
from jax.experimental.pallas import tpu_sc as plsc

N = 99999
E = 6399936
EPAD = 6400000
PER_W = 200000
CHUNK = 4000
NCHUNK = PER_W // CHUNK
NVEC = CHUNK // 16
UNROLL = 10
NPACK = 50000
NOUT = 33333
ACC = 33536
COLS = ACC // 16
NCV = COLS // 16


def _sc_body(xp_hbm, src_hbm, dst_hbm, w_hbm, part_hbm,
             xp_v, src_v, dst_v, acc_v, w_v, out_v, shr):
    cid = lax.axis_index("c")
    sid = lax.axis_index("s")
    wid = sid * 2 + cid

    pltpu.sync_copy(xp_hbm, xp_v)
    pltpu.sync_copy(w_hbm, w_v)

    zero = jnp.zeros((16,), jnp.float32)

    def zbody(j, carry):
        for u in range(16):
            acc_v[pl.ds((j * 16 + u) * 16, 16)] = zero
        return carry
    lax.fori_loop(0, COLS // 16, zbody, 0)

    ebase = wid * PER_W

    def chunk_body(g, carry):
        base = ebase + g * CHUNK
        pltpu.sync_copy(src_hbm.at[pl.ds(base, CHUNK)], src_v)
        pltpu.sync_copy(dst_hbm.at[pl.ds(base, CHUNK)], dst_v)

        def inner(i, c2):
            for u in range(UNROLL):
                o = (i * UNROLL + u) * 16
                s16 = src_v[pl.ds(o, 16)]
                d16 = dst_v[pl.ds(o, 16)]
                pk = plsc.load_gather(xp_v, [s16 >> 1])
                bits = jnp.where((s16 & 1) == 1, pk & jnp.int32(-65536),
                                 pk << 16)
                val = plsc.bitcast(bits, jnp.float32)
                q = lax.div(d16, jnp.int32(3))
                msk = (d16 - q * 3) == 0
                plsc.addupdate_scatter(acc_v, [q], val, mask=msk)
            return c2
        lax.fori_loop(0, NVEC // UNROLL, inner, 0)
        return carry
    lax.fori_loop(0, NCHUNK, chunk_body, 0)

    pltpu.sync_copy(acc_v, shr.at[pl.ds(sid * ACC, ACC)])
    plsc.subcore_barrier()

    colbase = sid * COLS
    for p in range(16):
        pltpu.sync_copy(shr.at[pl.ds(p * ACC + colbase, COLS)],
                        acc_v.at[pl.ds(p * COLS, COLS)])

    wv = w_v[...]

    def rbody(j, carry):
        o = j * 16
        t = acc_v[pl.ds(o, 16)]
        for p in range(1, 16):
            t = t + acc_v[pl.ds(p * COLS + o, 16)]
        out_v[pl.ds(o, 16)] = t * wv
        return carry
    lax.fori_loop(0, NCV, rbody, 0)

    pltpu.sync_copy(out_v, part_hbm.at[pl.ds(cid * ACC + colbase, COLS)])


def _combine_body(p_ref, o_ref):
    o_ref[...] = p_ref[:ACC] + p_ref[ACC:]


def kernel(x, edge_index, W):
    xb = x.reshape(-1).astype(jnp.bfloat16)
    xb = jnp.concatenate([xb, jnp.zeros((1,), jnp.bfloat16)])
    xp = lax.bitcast_convert_type(xb.reshape(NPACK, 2), jnp.int32)

    pad = EPAD - E
    src = jnp.concatenate([edge_index[0], jnp.zeros((pad,), jnp.int32)])
    dst = jnp.concatenate([edge_index[1], jnp.ones((pad,), jnp.int32)])
    wvec = jnp.broadcast_to(W.reshape(()), (16,)).astype(jnp.float32)

    mesh = plsc.VectorSubcoreMesh(core_axis_name="c", subcore_axis_name="s",
                                  num_cores=2, num_subcores=16)
    part = pl.kernel(
        _sc_body,
        out_type=jax.ShapeDtypeStruct((2 * ACC,), jnp.float32),
        mesh=mesh,
        compiler_params=pltpu.CompilerParams(needs_layout_passes=False),
        scratch_types=[
            pltpu.VMEM((NPACK,), jnp.int32),
            pltpu.VMEM((CHUNK,), jnp.int32),
            pltpu.VMEM((CHUNK,), jnp.int32),
            pltpu.VMEM((ACC,), jnp.float32),
            pltpu.VMEM((16,), jnp.float32),
            pltpu.VMEM((COLS,), jnp.float32),
            pltpu.VMEM_SHARED((16 * ACC,), jnp.float32),
        ],
    )(xp, src, dst, wvec)

    out = pl.pallas_call(
        _combine_body,
        out_shape=jax.ShapeDtypeStruct((ACC,), jnp.float32),
    )(part)
    return out[:NOUT]


# speedup vs baseline: 92.8019x; 1.0281x over previous
import jax
import jax.numpy as jnp
from jax import lax
from jax.experimental import pallas as pl
from jax.experimental.pallas import tpu as pltpu
from jax.experimental.pallas import tpu_sc as plsc

N = 99999
E = 6399936
EPAD = 6400000
PER_W = 200000
CHUNK = 4000
NCHUNK = PER_W // CHUNK
NVEC = CHUNK // 16
UNROLL = 10
NPACK = 50000
NOUT = 33333
ACC = 33536
COLS = ACC // 16
NCV = COLS // 16


def _sc_body(xp_hbm, src_hbm, dst_hbm, w_hbm, part_hbm,
             xp_v, src_v, dst_v, acc_v, w_v, out_v, shr):
    cid = lax.axis_index("c")
    sid = lax.axis_index("s")
    wid = sid * 2 + cid

    pltpu.sync_copy(xp_hbm, xp_v)
    pltpu.sync_copy(w_hbm, w_v)

    zero = jnp.zeros((16,), jnp.float32)

    def zbody(j, carry):
        for u in range(16):
            acc_v[pl.ds((j * 16 + u) * 16, 16)] = zero
        return carry
    lax.fori_loop(0, COLS // 16, zbody, 0)

    ebase = wid * PER_W

    def chunk_body(g, carry):
        base = ebase + g * CHUNK
        pltpu.sync_copy(src_hbm.at[pl.ds(base, CHUNK)], src_v)
        pltpu.sync_copy(dst_hbm.at[pl.ds(base, CHUNK)], dst_v)

        def inner(i, c2):
            for u in range(UNROLL):
                o = (i * UNROLL + u) * 16
                s16 = src_v[pl.ds(o, 16)]
                d16 = dst_v[pl.ds(o, 16)]
                pk = plsc.load_gather(xp_v, [s16 >> 1])
                bits = jnp.where((s16 & 1) == 1, pk & jnp.int32(-65536),
                                 pk << 16)
                val = plsc.bitcast(bits, jnp.float32)
                q = lax.div(d16, jnp.int32(3))
                msk = (d16 - q * 3) == 0
                val = jnp.where(msk, val, 0.0)
                plsc.addupdate_scatter(acc_v, [q], val)
            return c2
        lax.fori_loop(0, NVEC // UNROLL, inner, 0)
        return carry
    lax.fori_loop(0, NCHUNK, chunk_body, 0)

    pltpu.sync_copy(acc_v, shr.at[pl.ds(sid * ACC, ACC)])
    plsc.subcore_barrier()

    colbase = sid * COLS
    for p in range(16):
        pltpu.sync_copy(shr.at[pl.ds(p * ACC + colbase, COLS)],
                        acc_v.at[pl.ds(p * COLS, COLS)])

    wv = w_v[...]

    def rbody(j, carry):
        o = j * 16
        t = acc_v[pl.ds(o, 16)]
        for p in range(1, 16):
            t = t + acc_v[pl.ds(p * COLS + o, 16)]
        out_v[pl.ds(o, 16)] = t * wv
        return carry
    lax.fori_loop(0, NCV, rbody, 0)

    pltpu.sync_copy(out_v, part_hbm.at[pl.ds(cid * ACC + colbase, COLS)])


def _combine_body(p_ref, o_ref):
    o_ref[...] = p_ref[:ACC] + p_ref[ACC:]


def kernel(x, edge_index, W):
    xb = x.reshape(-1).astype(jnp.bfloat16)
    xb = jnp.concatenate([xb, jnp.zeros((1,), jnp.bfloat16)])
    xp = lax.bitcast_convert_type(xb.reshape(NPACK, 2), jnp.int32)

    pad = EPAD - E
    src = jnp.concatenate([edge_index[0], jnp.zeros((pad,), jnp.int32)])
    dst = jnp.concatenate([edge_index[1], jnp.ones((pad,), jnp.int32)])
    wvec = jnp.broadcast_to(W.reshape(()), (16,)).astype(jnp.float32)

    mesh = plsc.VectorSubcoreMesh(core_axis_name="c", subcore_axis_name="s",
                                  num_cores=2, num_subcores=16)
    part = pl.kernel(
        _sc_body,
        out_type=jax.ShapeDtypeStruct((2 * ACC,), jnp.float32),
        mesh=mesh,
        compiler_params=pltpu.CompilerParams(needs_layout_passes=False),
        scratch_types=[
            pltpu.VMEM((NPACK,), jnp.int32),
            pltpu.VMEM((CHUNK,), jnp.int32),
            pltpu.VMEM((CHUNK,), jnp.int32),
            pltpu.VMEM((ACC,), jnp.float32),
            pltpu.VMEM((16,), jnp.float32),
            pltpu.VMEM((COLS,), jnp.float32),
            pltpu.VMEM_SHARED((16 * ACC,), jnp.float32),
        ],
    )(xp, src, dst, wvec)

    out = pl.pallas_call(
        _combine_body,
        out_shape=jax.ShapeDtypeStruct((ACC,), jnp.float32),
    )(part)
    return out[:NOUT]


# speedup vs baseline: 95.2027x; 1.0259x over previous
import jax
import jax.numpy as jnp
from jax import lax
from jax.experimental import pallas as pl
from jax.experimental.pallas import tpu as pltpu
from jax.experimental.pallas import tpu_sc as plsc

N = 99999
E = 6399936
EPAD = 6400000
PER_W = 200000
CHUNK = 4000
NCHUNK = PER_W // CHUNK
NVEC = CHUNK // 16
UNROLL = 10
NPACK = 50000
NOUT = 33333
ACC = 33536
COLS = ACC // 16
NCV = COLS // 16


def _sc_body(xp_hbm, src_hbm, dst_hbm, w_hbm, part_hbm,
             xp_v, src0, dst0, src1, dst1, qb0, vb0, qb1, vb1,
             w_v, out_v, sem0, sem1, shr):
    cid = lax.axis_index("c")
    sid = lax.axis_index("s")
    wid = sid * 2 + cid

    pltpu.sync_copy(xp_hbm, xp_v)
    pltpu.sync_copy(w_hbm, w_v)

    zero = jnp.zeros((16,), jnp.float32)

    def zbody(j, carry):
        out_v[pl.ds(j * 16, 16)] = zero
        return carry
    lax.fori_loop(0, NCV, zbody, 0)
    pltpu.sync_copy(out_v, shr.at[pl.ds(sid * COLS, COLS)])
    plsc.subcore_barrier()

    ebase = wid * PER_W
    bufs = ((src0, dst0, qb0, vb0, sem0), (src1, dst1, qb1, vb1, sem1))

    def chunk_body(t, carry):
        for b, (srcv, dstv, qb, vb, sem) in enumerate(bufs):
            base = ebase + (t * 2 + b) * CHUNK
            pltpu.sync_copy(src_hbm.at[pl.ds(base, CHUNK)], srcv)
            pltpu.sync_copy(dst_hbm.at[pl.ds(base, CHUNK)], dstv)

            @pl.when(t > 0)
            def _wait_prev():
                pltpu.make_async_copy(vb, shr.at[qb], sem).wait()

            def inner(i, c2):
                for u in range(UNROLL):
                    o = (i * UNROLL + u) * 16
                    s16 = srcv[pl.ds(o, 16)]
                    d16 = dstv[pl.ds(o, 16)]
                    pk = plsc.load_gather(xp_v, [s16 >> 1])
                    bits = jnp.where((s16 & 1) == 1, pk & jnp.int32(-65536),
                                     pk << 16)
                    val = plsc.bitcast(bits, jnp.float32)
                    q = lax.div(d16, jnp.int32(3))
                    val = jnp.where((d16 - q * 3) == 0, val, 0.0)
                    qb[pl.ds(o, 16)] = q
                    vb[pl.ds(o, 16)] = val
                return c2
            lax.fori_loop(0, NVEC // UNROLL, inner, 0)
            pltpu.async_copy(vb, shr.at[qb], sem, add=True)
        return carry
    lax.fori_loop(0, NCHUNK // 2, chunk_body, 0)

    pltpu.make_async_copy(vb0, shr.at[qb0], sem0).wait()
    pltpu.make_async_copy(vb1, shr.at[qb1], sem1).wait()
    plsc.subcore_barrier()

    pltpu.sync_copy(shr.at[pl.ds(sid * COLS, COLS)], out_v)
    wv = w_v[...]

    def rbody(j, carry):
        o = j * 16
        out_v[pl.ds(o, 16)] = out_v[pl.ds(o, 16)] * wv
        return carry
    lax.fori_loop(0, NCV, rbody, 0)

    pltpu.sync_copy(out_v, part_hbm.at[pl.ds(cid * ACC + sid * COLS, COLS)])


def _combine_body(p_ref, o_ref):
    o_ref[...] = p_ref[:ACC] + p_ref[ACC:]


def kernel(x, edge_index, W):
    xb = x.reshape(-1).astype(jnp.bfloat16)
    xb = jnp.concatenate([xb, jnp.zeros((1,), jnp.bfloat16)])
    xp = lax.bitcast_convert_type(xb.reshape(NPACK, 2), jnp.int32)

    pad = EPAD - E
    src = jnp.concatenate([edge_index[0], jnp.zeros((pad,), jnp.int32)])
    dst = jnp.concatenate([edge_index[1], jnp.ones((pad,), jnp.int32)])
    wvec = jnp.broadcast_to(W.reshape(()), (16,)).astype(jnp.float32)

    mesh = plsc.VectorSubcoreMesh(core_axis_name="c", subcore_axis_name="s",
                                  num_cores=2, num_subcores=16)
    part = pl.kernel(
        _sc_body,
        out_type=jax.ShapeDtypeStruct((2 * ACC,), jnp.float32),
        mesh=mesh,
        compiler_params=pltpu.CompilerParams(needs_layout_passes=False),
        scratch_types=[
            pltpu.VMEM((NPACK,), jnp.int32),
            pltpu.VMEM((CHUNK,), jnp.int32),
            pltpu.VMEM((CHUNK,), jnp.int32),
            pltpu.VMEM((CHUNK,), jnp.int32),
            pltpu.VMEM((CHUNK,), jnp.int32),
            pltpu.VMEM((CHUNK,), jnp.int32),
            pltpu.VMEM((CHUNK,), jnp.float32),
            pltpu.VMEM((CHUNK,), jnp.int32),
            pltpu.VMEM((CHUNK,), jnp.float32),
            pltpu.VMEM((16,), jnp.float32),
            pltpu.VMEM((COLS,), jnp.float32),
            pltpu.SemaphoreType.DMA,
            pltpu.SemaphoreType.DMA,
            pltpu.VMEM_SHARED((ACC,), jnp.float32),
        ],
    )(xp, src, dst, wvec)

    out = pl.pallas_call(
        _combine_body,
        out_shape=jax.ShapeDtypeStruct((ACC,), jnp.float32),
    )(part)
    return out[:NOUT]


# speedup vs baseline: 111.5489x; 1.1717x over previous
import jax
import jax.numpy as jnp
from jax import lax
from jax.experimental import pallas as pl
from jax.experimental.pallas import tpu as pltpu
from jax.experimental.pallas import tpu_sc as plsc

N = 99999
E = 6399936
EPAD = 6400000
PER_W = 200000
CHUNK = 4000
NCHUNK = PER_W // CHUNK
NVEC = CHUNK // 16
UNROLL = 8
NPACK = 50000
NOUT = 33333
ACC = 33536
COLS = ACC // 16
NCV = COLS // 16


def _sc_body(xp_hbm, src_hbm, dst_hbm, w_hbm, part_hbm,
             xp_v, src_v, dst_v, acc_v, w_v, out_v, shr):
    cid = lax.axis_index("c")
    sid = lax.axis_index("s")
    wid = sid * 2 + cid

    pltpu.sync_copy(xp_hbm, xp_v)
    pltpu.sync_copy(w_hbm, w_v)

    zero = jnp.zeros((16,), jnp.float32)

    @plsc.parallel_loop(0, COLS, unroll=8)
    def _zero(j):
        acc_v[pl.ds(j * 16, 16)] = zero

    ebase = wid * PER_W

    def chunk_body(g, carry):
        base = ebase + g * CHUNK
        pltpu.sync_copy(src_hbm.at[pl.ds(base, CHUNK)], src_v)
        pltpu.sync_copy(dst_hbm.at[pl.ds(base, CHUNK)], dst_v)

        @plsc.parallel_loop(0, NVEC, unroll=UNROLL)
        def inner(i):
            o = i * 16
            s16 = src_v[pl.ds(o, 16)]
            d16 = dst_v[pl.ds(o, 16)]
            pk = plsc.load_gather(xp_v, [s16 >> 1])
            bits = jnp.where((s16 & 1) == 1, pk & jnp.int32(-65536),
                             pk << 16)
            val = plsc.bitcast(bits, jnp.float32)
            q = lax.div(d16, jnp.int32(3))
            val = jnp.where((d16 - q * 3) == 0, val, 0.0)
            plsc.addupdate_scatter(acc_v, [q], val)
        return carry
    lax.fori_loop(0, NCHUNK, chunk_body, 0)

    pltpu.sync_copy(acc_v, shr.at[pl.ds(sid * ACC, ACC)])
    plsc.subcore_barrier()

    colbase = sid * COLS
    for p in range(16):
        pltpu.sync_copy(shr.at[pl.ds(p * ACC + colbase, COLS)],
                        acc_v.at[pl.ds(p * COLS, COLS)])

    wv = w_v[...]

    @plsc.parallel_loop(0, NCV, unroll=2)
    def rbody(j):
        o = j * 16
        t = acc_v[pl.ds(o, 16)]
        for p in range(1, 16):
            t = t + acc_v[pl.ds(p * COLS + o, 16)]
        out_v[pl.ds(o, 16)] = t * wv

    pltpu.sync_copy(out_v, part_hbm.at[pl.ds(cid * ACC + colbase, COLS)])


def _combine_body(p_ref, o_ref):
    o_ref[...] = p_ref[:ACC] + p_ref[ACC:]


def kernel(x, edge_index, W):
    xb = x.reshape(-1).astype(jnp.bfloat16)
    xb = jnp.concatenate([xb, jnp.zeros((1,), jnp.bfloat16)])
    xp = lax.bitcast_convert_type(xb.reshape(NPACK, 2), jnp.int32)

    pad = EPAD - E
    src = jnp.concatenate([edge_index[0], jnp.zeros((pad,), jnp.int32)])
    dst = jnp.concatenate([edge_index[1], jnp.ones((pad,), jnp.int32)])
    wvec = jnp.broadcast_to(W.reshape(()), (16,)).astype(jnp.float32)

    mesh = plsc.VectorSubcoreMesh(core_axis_name="c", subcore_axis_name="s",
                                  num_cores=2, num_subcores=16)
    part = pl.kernel(
        _sc_body,
        out_type=jax.ShapeDtypeStruct((2 * ACC,), jnp.float32),
        mesh=mesh,
        compiler_params=pltpu.CompilerParams(needs_layout_passes=False),
        scratch_types=[
            pltpu.VMEM((NPACK,), jnp.int32),
            pltpu.VMEM((CHUNK,), jnp.int32),
            pltpu.VMEM((CHUNK,), jnp.int32),
            pltpu.VMEM((ACC,), jnp.float32),
            pltpu.VMEM((16,), jnp.float32),
            pltpu.VMEM((COLS,), jnp.float32),
            pltpu.VMEM_SHARED((16 * ACC,), jnp.float32),
        ],
    )(xp, src, dst, wvec)

    out = pl.pallas_call(
        _combine_body,
        out_shape=jax.ShapeDtypeStruct((ACC,), jnp.float32),
    )(part)
    return out[:NOUT]
